# Initial kernel scaffold; baseline (speedup 1.0000x reference)
import jax
import jax.numpy as jnp
from jax.experimental import pallas as pl


def kernel(x, edge_index, W0, b0, W1, b1, W2, b2, W3, b3):
    raise NotImplementedError("write your pallas kernel here")


# speedup vs baseline: 17.7791x; 17.7791x over previous
import functools

import jax
import jax.numpy as jnp
from jax import lax
from jax.experimental import pallas as pl
from jax.experimental.pallas import tpu as pltpu
from jax.experimental.pallas import tpu_sc as plsc

N = 10000
D = 128
NC = 2
NS = 16
NT = NC * NS
CH = 128
CPT = 80
EPT = CPT * CH
EP = NT * EPT
NPAD = 10240
RPS = NPAD // NS

_mesh = plsc.VectorSubcoreMesh(core_axis_name="c", subcore_axis_name="s")
_sc_params = pltpu.CompilerParams(needs_layout_passes=False)


@functools.partial(
    pl.kernel,
    mesh=_mesh,
    out_type=jax.ShapeDtypeStruct((NT, NPAD), jnp.float32),
    compiler_params=_sc_params,
    scratch_types=[
        pltpu.VMEM((EPT,), jnp.int32),
        pltpu.VMEM((NPAD,), jnp.float32),
    ],
)
def _deg_kernel(dst_hbm, out_hbm, dst_v, hist_v):
    c = lax.axis_index("c")
    s = lax.axis_index("s")
    t = c * NS + s
    pltpu.sync_copy(dst_hbm.at[t], dst_v)

    def zero_body(i, carry):
        hist_v[pl.ds(i * 16, 16)] = jnp.zeros((16,), jnp.float32)
        return carry

    lax.fori_loop(0, NPAD // 16, zero_body, 0)

    ones = jnp.ones((16,), jnp.float32)

    def body(i, carry):
        idx = dst_v[pl.ds(i * 16, 16)]
        plsc.addupdate_scatter(hist_v, [idx], ones)
        return carry

    lax.fori_loop(0, EPT // 16, body, 0)
    pltpu.sync_copy(hist_v, out_hbm.at[t])


@functools.partial(
    pl.kernel,
    mesh=_mesh,
    out_type=jax.ShapeDtypeStruct((NC, NPAD, D), jnp.float32),
    compiler_params=_sc_params,
    scratch_types=[
        pltpu.VMEM((CPT, CH), jnp.int32),
        pltpu.VMEM((CPT, CH), jnp.int32),
        pltpu.VMEM((CH, D), jnp.float32),
        pltpu.VMEM_SHARED((NPAD, D), jnp.float32),
        pltpu.SemaphoreType.DMA,
    ],
)
def _edge_kernel(h_hbm, src_hbm, dst_hbm, out_hbm, src_v, dst_v, rows_v, acc, sem):
    c = lax.axis_index("c")
    s = lax.axis_index("s")
    t = c * NS + s
    pltpu.sync_copy(src_hbm.at[t], src_v)
    pltpu.sync_copy(dst_hbm.at[t], dst_v)

    def zbody(i, carry):
        r = i // 8
        j = i % 8
        rows_v[r, pl.ds(j * 16, 16)] = jnp.zeros((16,), jnp.float32)
        return carry

    lax.fori_loop(0, CH * 8, zbody, 0)

    def zcopy(k, carry):
        pltpu.sync_copy(rows_v, acc.at[pl.ds(s * RPS + k * CH, CH)])
        return carry

    lax.fori_loop(0, RPS // CH, zcopy, 0)
    plsc.subcore_barrier()

    def body(g, carry):
        pltpu.async_copy(h_hbm.at[src_v.at[g]], rows_v, sem).wait()
        pltpu.sync_copy(rows_v, acc.at[dst_v.at[g]], add=True)
        return carry

    lax.fori_loop(0, CPT, body, 0)
    plsc.subcore_barrier()

    def ocopy(k, carry):
        pltpu.sync_copy(
            acc.at[pl.ds(s * RPS + k * CH, CH)],
            out_hbm.at[c, pl.ds(s * RPS + k * CH, CH)],
        )
        return carry

    lax.fori_loop(0, RPS // CH, ocopy, 0)


def _tc0_body(hists_ref, x_ref, w_ref, h_ref, dis_ref):
    deg = jnp.sum(hists_ref[:, :N], axis=0) + 1.0
    dis = lax.rsqrt(deg)[:, None]
    dis_ref[...] = dis
    h_ref[...] = jnp.dot(
        x_ref[...] * dis, w_ref[...], preferred_element_type=jnp.float32
    )


def _tc0(hists, x, w):
    return pl.pallas_call(
        _tc0_body,
        out_shape=(
            jax.ShapeDtypeStruct((N, D), jnp.float32),
            jax.ShapeDtypeStruct((N, 1), jnp.float32),
        ),
    )(hists, x, w)


def _fuse_body(p_ref, h_ref, dis_ref, b_ref, w_ref, o_ref):
    dis = dis_ref[...]
    a = p_ref[0, :N] + p_ref[1, :N] + h_ref[...]
    x = jnp.maximum(a * dis + b_ref[...], 0.0)
    o_ref[...] = jnp.dot(x * dis, w_ref[...], preferred_element_type=jnp.float32)


def _fuse(p, h, dis, b, w):
    return pl.pallas_call(
        _fuse_body,
        out_shape=jax.ShapeDtypeStruct((N, D), jnp.float32),
    )(p, h, dis, b, w)


def _final_body(p_ref, h_ref, dis_ref, b_ref, o_ref):
    a = p_ref[0, :N] + p_ref[1, :N] + h_ref[...]
    o_ref[...] = a * dis_ref[...] + b_ref[...]


def _final(p, h, dis, b):
    return pl.pallas_call(
        _final_body,
        out_shape=jax.ShapeDtypeStruct((N, D), jnp.float32),
    )(p, h, dis, b)


def kernel(x, edge_index, W0, b0, W1, b1, W2, b2, W3, b3):
    src = edge_index[0].astype(jnp.int32)
    dst = edge_index[1].astype(jnp.int32)
    e = src.shape[0]
    pad_n = EP - e
    ar = jnp.arange(pad_n, dtype=jnp.int32)
    src_p = jnp.concatenate([src, ar % N]).reshape(NT, CPT, CH)
    dst_p = jnp.concatenate([dst, N + ar % (NPAD - N)]).reshape(NT, CPT, CH)
    dst_flat = dst_p.reshape(NT, EPT)

    hists = _deg_kernel(dst_flat)
    h, dis = _tc0(hists, x, W0)
    b_prev = [b0, b1, b2]
    w_next = [W1, W2, W3]
    for i in range(3):
        p = _edge_kernel(h, src_p, dst_p)
        h = _fuse(p, h, dis, b_prev[i].reshape(1, D), w_next[i])
    p = _edge_kernel(h, src_p, dst_p)
    return _final(p, h, dis, b3.reshape(1, D))


# speedup vs baseline: 26.6432x; 1.4986x over previous
import functools

import jax
import jax.numpy as jnp
from jax import lax
from jax.experimental import pallas as pl
from jax.experimental.pallas import tpu as pltpu
from jax.experimental.pallas import tpu_sc as plsc

N = 10000
D = 128
NC = 2
NS = 16
NT = NC * NS
CH = 128
CPT = 80
EPT = CPT * CH
EP = NT * EPT
NPAD = 10240
RPS = NPAD // NS

_mesh = plsc.VectorSubcoreMesh(core_axis_name="c", subcore_axis_name="s")
_sc_params = pltpu.CompilerParams(needs_layout_passes=False)


@functools.partial(
    pl.kernel,
    mesh=_mesh,
    out_type=jax.ShapeDtypeStruct((NT, NPAD), jnp.float32),
    compiler_params=_sc_params,
    scratch_types=[
        pltpu.VMEM((EPT,), jnp.int32),
        pltpu.VMEM((NPAD,), jnp.float32),
    ],
)
def _deg_kernel(dst_hbm, out_hbm, dst_v, hist_v):
    c = lax.axis_index("c")
    s = lax.axis_index("s")
    t = c * NS + s
    pltpu.sync_copy(dst_hbm.at[t], dst_v)

    def zero_body(i, carry):
        hist_v[pl.ds(i * 16, 16)] = jnp.zeros((16,), jnp.float32)
        return carry

    lax.fori_loop(0, NPAD // 16, zero_body, 0)

    ones = jnp.ones((16,), jnp.float32)

    def body(i, carry):
        idx = dst_v[pl.ds(i * 16, 16)]
        plsc.addupdate_scatter(hist_v, [idx], ones)
        return carry

    lax.fori_loop(0, EPT // 16, body, 0)
    pltpu.sync_copy(hist_v, out_hbm.at[t])


@functools.partial(
    pl.kernel,
    mesh=_mesh,
    out_type=jax.ShapeDtypeStruct((NC, NPAD, D), jnp.float32),
    compiler_params=_sc_params,
    scratch_types=[
        pltpu.VMEM((CPT // 2, CH), jnp.int32),
        pltpu.VMEM((CPT // 2, CH), jnp.int32),
        pltpu.VMEM((CH, D), jnp.float32),
        pltpu.VMEM((CH, D), jnp.float32),
        pltpu.VMEM_SHARED((NPAD, D), jnp.float32),
        pltpu.SemaphoreType.DMA,
        pltpu.SemaphoreType.DMA,
    ],
)
def _edge_kernel(
    h_hbm, src_hbm, dst_hbm, out_hbm, src_v, dst_v, rows_a, rows_b, acc, sem_a, sem_b
):
    c = lax.axis_index("c")
    s = lax.axis_index("s")
    t = c * NS + s
    hcpt = CPT // 2

    def zbody(i, carry):
        r = i // 8
        j = i % 8
        rows_a[r, pl.ds(j * 16, 16)] = jnp.zeros((16,), jnp.float32)
        return carry

    lax.fori_loop(0, CH * 8, zbody, 0)

    def zcopy(k, carry):
        pltpu.sync_copy(rows_a, acc.at[pl.ds(s * RPS + k * CH, CH)])
        return carry

    lax.fori_loop(0, RPS // CH, zcopy, 0)
    plsc.subcore_barrier()

    npair = hcpt // 2
    for half in range(2):
        pltpu.sync_copy(src_hbm.at[t, pl.ds(half * hcpt, hcpt)], src_v)
        pltpu.sync_copy(dst_hbm.at[t, pl.ds(half * hcpt, hcpt)], dst_v)
        pltpu.async_copy(h_hbm.at[src_v.at[0]], rows_a, sem_a)

        def body(k, carry):
            g0 = 2 * k
            g1 = g0 + 1
            pltpu.async_copy(h_hbm.at[src_v.at[g1]], rows_b, sem_b)
            pltpu.make_async_copy(h_hbm.at[src_v.at[g0]], rows_a, sem_a).wait()
            pltpu.sync_copy(rows_a, acc.at[dst_v.at[g0]], add=True)

            @pl.when(k < npair - 1)
            def _():
                pltpu.async_copy(h_hbm.at[src_v.at[g0 + 2]], rows_a, sem_a)

            pltpu.make_async_copy(h_hbm.at[src_v.at[g1]], rows_b, sem_b).wait()
            pltpu.sync_copy(rows_b, acc.at[dst_v.at[g1]], add=True)
            return carry

        lax.fori_loop(0, npair, body, 0)
    plsc.subcore_barrier()

    def ocopy(k, carry):
        pltpu.sync_copy(
            acc.at[pl.ds(s * RPS + k * CH, CH)],
            out_hbm.at[c, pl.ds(s * RPS + k * CH, CH)],
        )
        return carry

    lax.fori_loop(0, RPS // CH, ocopy, 0)


def _tc0_body(hists_ref, x_ref, w_ref, h_ref, dis_ref):
    deg = jnp.sum(hists_ref[:, :N], axis=0) + 1.0
    dis = lax.rsqrt(deg)[:, None]
    dis_ref[...] = dis
    h_ref[...] = jnp.dot(
        x_ref[...] * dis, w_ref[...], preferred_element_type=jnp.float32
    )


def _tc0(hists, x, w):
    return pl.pallas_call(
        _tc0_body,
        out_shape=(
            jax.ShapeDtypeStruct((N, D), jnp.float32),
            jax.ShapeDtypeStruct((N, 1), jnp.float32),
        ),
    )(hists, x, w)


def _fuse_body(p_ref, h_ref, dis_ref, b_ref, w_ref, o_ref):
    dis = dis_ref[...]
    a = p_ref[0, :N] + p_ref[1, :N] + h_ref[...]
    x = jnp.maximum(a * dis + b_ref[...], 0.0)
    o_ref[...] = jnp.dot(x * dis, w_ref[...], preferred_element_type=jnp.float32)


def _fuse(p, h, dis, b, w):
    return pl.pallas_call(
        _fuse_body,
        out_shape=jax.ShapeDtypeStruct((N, D), jnp.float32),
    )(p, h, dis, b, w)


def _final_body(p_ref, h_ref, dis_ref, b_ref, o_ref):
    a = p_ref[0, :N] + p_ref[1, :N] + h_ref[...]
    o_ref[...] = a * dis_ref[...] + b_ref[...]


def _final(p, h, dis, b):
    return pl.pallas_call(
        _final_body,
        out_shape=jax.ShapeDtypeStruct((N, D), jnp.float32),
    )(p, h, dis, b)


def kernel(x, edge_index, W0, b0, W1, b1, W2, b2, W3, b3):
    src = edge_index[0].astype(jnp.int32)
    dst = edge_index[1].astype(jnp.int32)
    e = src.shape[0]
    pad_n = EP - e
    ar = jnp.arange(pad_n, dtype=jnp.int32)
    src_p = jnp.concatenate([src, ar % N]).reshape(NT, CPT, CH)
    dst_p = jnp.concatenate([dst, N + ar % (NPAD - N)]).reshape(NT, CPT, CH)
    dst_flat = dst_p.reshape(NT, EPT)

    hists = _deg_kernel(dst_flat)
    h, dis = _tc0(hists, x, W0)
    b_prev = [b0, b1, b2]
    w_next = [W1, W2, W3]
    for i in range(3):
        p = _edge_kernel(h, src_p, dst_p)
        h = _fuse(p, h, dis, b_prev[i].reshape(1, D), w_next[i])
    p = _edge_kernel(h, src_p, dst_p)
    return _final(p, h, dis, b3.reshape(1, D))
